# Initial kernel scaffold; baseline (speedup 1.0000x reference)
import functools
import jax
import jax.numpy as jnp
from jax import lax
from jax.experimental import pallas as pl
from jax.experimental.pallas import tpu as pltpu
from jax.experimental.pallas import tpu_sc as plsc

NC = 2
NS = 16
NW = NC * NS

N = 10000
E = 320000
H = 128
EA_W = 32
EPW = E // NW
K = 80
NCHUNK = EPW // K
RPT = N // NS

_mesh = plsc.VectorSubcoreMesh(core_axis_name="c", subcore_axis_name="s")


@functools.partial(
    pl.kernel,
    out_type=jax.ShapeDtypeStruct((NC, N, EA_W), jnp.float32),
    mesh=_mesh,
    scratch_types=[
        pltpu.VMEM_SHARED((N, EA_W), jnp.float32),
        pltpu.VMEM((K,), jnp.int32),
        pltpu.VMEM((K, EA_W), jnp.float32),
    ],
)
def _ea_deg_kernel(ea_hbm, dst_hbm, zeros_hbm, out_hbm, acc, didx, rows):
    cid = lax.axis_index("c")
    sid = lax.axis_index("s")
    wid = sid * NC + cid
    pltpu.sync_copy(zeros_hbm.at[pl.ds(sid * RPT, RPT)],
                    acc.at[pl.ds(sid * RPT, RPT)])
    plsc.subcore_barrier()

    def body(j, _):
        base = wid * EPW + j * K
        pltpu.sync_copy(dst_hbm.at[pl.ds(base, K)], didx)
        pltpu.sync_copy(ea_hbm.at[pl.ds(base, K)], rows)
        pltpu.sync_copy(rows, acc.at[didx], add=True)
        return 0

    lax.fori_loop(0, NCHUNK, body, 0)
    plsc.subcore_barrier()
    pltpu.sync_copy(acc.at[pl.ds(sid * RPT, RPT)],
                    out_hbm.at[cid].at[pl.ds(sid * RPT, RPT)])


@functools.partial(
    pl.kernel,
    out_type=jax.ShapeDtypeStruct((NC, N, H), jnp.float32),
    mesh=_mesh,
    scratch_types=[
        pltpu.VMEM_SHARED((N, H), jnp.float32),
        pltpu.VMEM((K,), jnp.int32),
        pltpu.VMEM((K,), jnp.int32),
        pltpu.VMEM((K, H), jnp.float32),
        pltpu.SemaphoreType.DMA,
    ],
)
def _agg_kernel(y_hbm, src_hbm, dst_hbm, zeros_hbm, out_hbm,
                acc, sidx, didx, rows, sem):
    cid = lax.axis_index("c")
    sid = lax.axis_index("s")
    wid = sid * NC + cid
    pltpu.sync_copy(zeros_hbm.at[pl.ds(sid * RPT, RPT)],
                    acc.at[pl.ds(sid * RPT, RPT)])
    plsc.subcore_barrier()

    def body(j, _):
        base = wid * EPW + j * K
        pltpu.sync_copy(src_hbm.at[pl.ds(base, K)], sidx)
        pltpu.sync_copy(dst_hbm.at[pl.ds(base, K)], didx)
        pltpu.async_copy(y_hbm.at[sidx], rows, sem).wait()
        pltpu.sync_copy(rows, acc.at[didx], add=True)
        return 0

    lax.fori_loop(0, NCHUNK, body, 0)
    plsc.subcore_barrier()
    pltpu.sync_copy(acc.at[pl.ds(sid * RPT, RPT)],
                    out_hbm.at[cid].at[pl.ds(sid * RPT, RPT)])


RB = 2000
NBLK = N // RB


def _mm_body(x_ref, w_ref, o_ref):
    o_ref[...] = jnp.dot(x_ref[...], w_ref[...],
                         preferred_element_type=jnp.float32)


_mm = pl.pallas_call(
    _mm_body,
    grid=(NBLK,),
    in_specs=[
        pl.BlockSpec((RB, H), lambda i: (i, 0)),
        pl.BlockSpec((H, H), lambda i: (0, 0)),
    ],
    out_specs=pl.BlockSpec((RB, H), lambda i: (i, 0)),
    out_shape=jax.ShapeDtypeStruct((N, H), jnp.float32),
)


def _combine_core(p_ref, x_ref, ea_ref, wiT_ref, weT_ref, b_ref):
    p = p_ref[0] + p_ref[1]
    ea = ea_ref[0] + ea_ref[1]
    deg = ea[:, 16:17]
    eagg = ea[:, :16]
    z = jnp.dot(x_ref[...], wiT_ref[...], preferred_element_type=jnp.float32)
    c = jnp.dot(eagg, weT_ref[...], preferred_element_type=jnp.float32)
    h = p + deg * (z + b_ref[...]) + c
    return jnp.where(h > 0, h, 0.01 * h)


def _combine_mid_body(p_ref, x_ref, ea_ref, wiT_ref, weT_ref, b_ref, wjT_ref,
                      xo_ref, yo_ref):
    xn = _combine_core(p_ref, x_ref, ea_ref, wiT_ref, weT_ref, b_ref)
    xo_ref[...] = xn
    yo_ref[...] = jnp.dot(xn, wjT_ref[...], preferred_element_type=jnp.float32)


def _combine_last_body(p_ref, x_ref, ea_ref, wiT_ref, weT_ref, b_ref, xo_ref):
    xn = _combine_core(p_ref, x_ref, ea_ref, wiT_ref, weT_ref, b_ref)
    xo_ref[...] = jnp.where(xn > 0, xn, 0.01 * xn)


_combine_in_specs = [
    pl.BlockSpec((NC, RB, H), lambda i: (0, i, 0)),
    pl.BlockSpec((RB, H), lambda i: (i, 0)),
    pl.BlockSpec((NC, RB, EA_W), lambda i: (0, i, 0)),
    pl.BlockSpec((H, H), lambda i: (0, 0)),
    pl.BlockSpec((16, H), lambda i: (0, 0)),
    pl.BlockSpec((1, H), lambda i: (0, 0)),
]

_combine_mid = pl.pallas_call(
    _combine_mid_body,
    grid=(NBLK,),
    in_specs=_combine_in_specs + [pl.BlockSpec((H, H), lambda i: (0, 0))],
    out_specs=[
        pl.BlockSpec((RB, H), lambda i: (i, 0)),
        pl.BlockSpec((RB, H), lambda i: (i, 0)),
    ],
    out_shape=[
        jax.ShapeDtypeStruct((N, H), jnp.float32),
        jax.ShapeDtypeStruct((N, H), jnp.float32),
    ],
)

_combine_last = pl.pallas_call(
    _combine_last_body,
    grid=(NBLK,),
    in_specs=_combine_in_specs,
    out_specs=pl.BlockSpec((RB, H), lambda i: (i, 0)),
    out_shape=jax.ShapeDtypeStruct((N, H), jnp.float32),
)


@jax.jit
def kernel(x, edge_index, edge_attr, Wm, bm):
    src = edge_index[0]
    dst = edge_index[1]
    ea_pad = jnp.concatenate(
        [edge_attr,
         jnp.ones((E, 1), jnp.float32),
         jnp.zeros((E, EA_W - 17), jnp.float32)], axis=1)
    zeros_h = jnp.zeros((N, H), jnp.float32)
    zeros_ea = jnp.zeros((N, EA_W), jnp.float32)

    ea_parts = _ea_deg_kernel(ea_pad, dst, zeros_ea)

    wjT = [Wm[l, :, :H].T for l in range(3)]
    wiT = [Wm[l, :, H:2 * H].T for l in range(3)]
    weT = [Wm[l, :, 2 * H:].T for l in range(3)]
    b = [bm[l].reshape(1, H) for l in range(3)]

    y = _mm(x, wjT[0])
    for l in range(3):
        parts = _agg_kernel(y, src, dst, zeros_h)
        if l < 2:
            x, y = _combine_mid(parts, x, ea_parts, wiT[l], weT[l], b[l],
                                wjT[l + 1])
        else:
            x = _combine_last(parts, x, ea_parts, wiT[l], weT[l], b[l])
    return x


# speedup vs baseline: 5.9909x; 5.9909x over previous
import functools
import jax
import jax.numpy as jnp
from jax import lax
from jax.experimental import pallas as pl
from jax.experimental.pallas import tpu as pltpu
from jax.experimental.pallas import tpu_sc as plsc

NC = 2
NS = 16
NW = NC * NS

N = 10000
E = 320000
H = 128
EA_W = 128
EPW = E // NW
K = 80
NCHUNK = EPW // K
RPT = 640
RPT_LAST = N - (NS - 1) * RPT


def _striped_copy(src_at, dst_at, sid):
    @pl.when(sid < NS - 1)
    def _():
        off = pl.multiple_of(sid * RPT, 8)
        pltpu.sync_copy(src_at(off, RPT), dst_at(off, RPT))

    @pl.when(sid == NS - 1)
    def _():
        pltpu.sync_copy(src_at((NS - 1) * RPT, RPT_LAST),
                        dst_at((NS - 1) * RPT, RPT_LAST))

_mesh = plsc.VectorSubcoreMesh(core_axis_name="c", subcore_axis_name="s")


@functools.partial(
    pl.kernel,
    out_type=jax.ShapeDtypeStruct((NC, N, EA_W), jnp.float32),
    mesh=_mesh,
    scratch_types=[
        pltpu.VMEM_SHARED((N, EA_W), jnp.float32),
        pltpu.VMEM((K,), jnp.int32),
        pltpu.VMEM((K, EA_W), jnp.float32),
    ],
)
def _ea_deg_kernel(ea_hbm, dst_hbm, zeros_hbm, out_hbm, acc, didx, rows):
    cid = lax.axis_index("c")
    sid = lax.axis_index("s")
    wid = sid * NC + cid
    _striped_copy(lambda o, s: zeros_hbm.at[pl.ds(o, s)],
                  lambda o, s: acc.at[pl.ds(o, s)], sid)
    plsc.subcore_barrier()

    def body(j, _):
        base = wid * EPW + j * K
        pltpu.sync_copy(dst_hbm.at[pl.ds(base, K)], didx)
        pltpu.sync_copy(ea_hbm.at[pl.ds(base, K)], rows)
        pltpu.sync_copy(rows, acc.at[didx], add=True)
        return 0

    lax.fori_loop(0, NCHUNK, body, 0)
    plsc.subcore_barrier()
    _striped_copy(lambda o, s: acc.at[pl.ds(o, s)],
                  lambda o, s: out_hbm.at[cid].at[pl.ds(o, s)], sid)


@functools.partial(
    pl.kernel,
    out_type=jax.ShapeDtypeStruct((NC, N, H), jnp.float32),
    mesh=_mesh,
    scratch_types=[
        pltpu.VMEM_SHARED((N, H), jnp.float32),
        pltpu.VMEM((K,), jnp.int32),
        pltpu.VMEM((K,), jnp.int32),
        pltpu.VMEM((K, H), jnp.float32),
        pltpu.SemaphoreType.DMA,
    ],
)
def _agg_kernel(y_hbm, src_hbm, dst_hbm, zeros_hbm, out_hbm,
                acc, sidx, didx, rows, sem):
    cid = lax.axis_index("c")
    sid = lax.axis_index("s")
    wid = sid * NC + cid
    _striped_copy(lambda o, s: zeros_hbm.at[pl.ds(o, s)],
                  lambda o, s: acc.at[pl.ds(o, s)], sid)
    plsc.subcore_barrier()

    def body(j, _):
        base = wid * EPW + j * K
        pltpu.sync_copy(src_hbm.at[pl.ds(base, K)], sidx)
        pltpu.sync_copy(dst_hbm.at[pl.ds(base, K)], didx)
        pltpu.async_copy(y_hbm.at[sidx], rows, sem).wait()
        pltpu.sync_copy(rows, acc.at[didx], add=True)
        return 0

    lax.fori_loop(0, NCHUNK, body, 0)
    plsc.subcore_barrier()
    _striped_copy(lambda o, s: acc.at[pl.ds(o, s)],
                  lambda o, s: out_hbm.at[cid].at[pl.ds(o, s)], sid)


RB = 2000
NBLK = N // RB


def _mm_body(x_ref, w_ref, o_ref):
    o_ref[...] = jnp.dot(x_ref[...], w_ref[...],
                         preferred_element_type=jnp.float32)


_mm = pl.pallas_call(
    _mm_body,
    grid=(NBLK,),
    in_specs=[
        pl.BlockSpec((RB, H), lambda i: (i, 0)),
        pl.BlockSpec((H, H), lambda i: (0, 0)),
    ],
    out_specs=pl.BlockSpec((RB, H), lambda i: (i, 0)),
    out_shape=jax.ShapeDtypeStruct((N, H), jnp.float32),
)


def _combine_core(p_ref, x_ref, ea_ref, wiT_ref, weT_ref, b_ref):
    p = p_ref[0] + p_ref[1]
    ea = ea_ref[0] + ea_ref[1]
    deg = ea[:, 16:17]
    eagg = ea[:, :16]
    z = jnp.dot(x_ref[...], wiT_ref[...], preferred_element_type=jnp.float32)
    c = jnp.dot(eagg, weT_ref[...], preferred_element_type=jnp.float32)
    h = p + deg * (z + b_ref[...]) + c
    return jnp.where(h > 0, h, 0.01 * h)


def _combine_mid_body(p_ref, x_ref, ea_ref, wiT_ref, weT_ref, b_ref, wjT_ref,
                      xo_ref, yo_ref):
    xn = _combine_core(p_ref, x_ref, ea_ref, wiT_ref, weT_ref, b_ref)
    xo_ref[...] = xn
    yo_ref[...] = jnp.dot(xn, wjT_ref[...], preferred_element_type=jnp.float32)


def _combine_last_body(p_ref, x_ref, ea_ref, wiT_ref, weT_ref, b_ref, xo_ref):
    xn = _combine_core(p_ref, x_ref, ea_ref, wiT_ref, weT_ref, b_ref)
    xo_ref[...] = jnp.where(xn > 0, xn, 0.01 * xn)


_combine_in_specs = [
    pl.BlockSpec((NC, RB, H), lambda i: (0, i, 0)),
    pl.BlockSpec((RB, H), lambda i: (i, 0)),
    pl.BlockSpec((NC, RB, EA_W), lambda i: (0, i, 0)),
    pl.BlockSpec((H, H), lambda i: (0, 0)),
    pl.BlockSpec((16, H), lambda i: (0, 0)),
    pl.BlockSpec((1, H), lambda i: (0, 0)),
]

_combine_mid = pl.pallas_call(
    _combine_mid_body,
    grid=(NBLK,),
    in_specs=_combine_in_specs + [pl.BlockSpec((H, H), lambda i: (0, 0))],
    out_specs=[
        pl.BlockSpec((RB, H), lambda i: (i, 0)),
        pl.BlockSpec((RB, H), lambda i: (i, 0)),
    ],
    out_shape=[
        jax.ShapeDtypeStruct((N, H), jnp.float32),
        jax.ShapeDtypeStruct((N, H), jnp.float32),
    ],
)

_combine_last = pl.pallas_call(
    _combine_last_body,
    grid=(NBLK,),
    in_specs=_combine_in_specs,
    out_specs=pl.BlockSpec((RB, H), lambda i: (i, 0)),
    out_shape=jax.ShapeDtypeStruct((N, H), jnp.float32),
)


@jax.jit
def kernel(x, edge_index, edge_attr, Wm, bm):
    src = edge_index[0]
    dst = edge_index[1]
    ea_pad = jnp.concatenate(
        [edge_attr,
         jnp.ones((E, 1), jnp.float32),
         jnp.zeros((E, EA_W - 17), jnp.float32)], axis=1)
    zeros_h = jnp.zeros((N, H), jnp.float32)
    zeros_ea = jnp.zeros((N, EA_W), jnp.float32)

    ea_parts = _ea_deg_kernel(ea_pad, dst, zeros_ea)

    wjT = [Wm[l, :, :H].T for l in range(3)]
    wiT = [Wm[l, :, H:2 * H].T for l in range(3)]
    weT = [Wm[l, :, 2 * H:].T for l in range(3)]
    b = [bm[l].reshape(1, H) for l in range(3)]

    y = _mm(x, wjT[0])
    for l in range(3):
        parts = _agg_kernel(y, src, dst, zeros_h)
        if l < 2:
            x, y = _combine_mid(parts, x, ea_parts, wiT[l], weT[l], b[l],
                                wjT[l + 1])
        else:
            x = _combine_last(parts, x, ea_parts, wiT[l], weT[l], b[l])
    return x


# speedup vs baseline: 11.6286x; 1.9411x over previous
import functools
import jax
import jax.numpy as jnp
from jax import lax
from jax.experimental import pallas as pl
from jax.experimental.pallas import tpu as pltpu
from jax.experimental.pallas import tpu_sc as plsc

NC = 2
NS = 16
NW = NC * NS

N = 10000
E = 320000
H = 128
EA_W = 128
K = 128
NCHUNK = E // K
TPW = NCHUNK // NW
NTAIL = NCHUNK - TPW * NW
RPT = 640
RPT_LAST = N - (NS - 1) * RPT


def _striped_copy(src_at, dst_at, sid):
    @pl.when(sid < NS - 1)
    def _():
        off = pl.multiple_of(sid * RPT, 8)
        pltpu.sync_copy(src_at(off, RPT), dst_at(off, RPT))

    @pl.when(sid == NS - 1)
    def _():
        pltpu.sync_copy(src_at((NS - 1) * RPT, RPT_LAST),
                        dst_at((NS - 1) * RPT, RPT_LAST))

_mesh = plsc.VectorSubcoreMesh(core_axis_name="c", subcore_axis_name="s")


NB = 3
NOUT = TPW // NB


@functools.partial(
    pl.kernel,
    out_type=jax.ShapeDtypeStruct((NC, N, EA_W), jnp.float32),
    mesh=_mesh,
    scratch_types=[
        pltpu.VMEM_SHARED((N, EA_W), jnp.float32),
        pltpu.VMEM((NB, K), jnp.int32),
        pltpu.VMEM((NB, K, EA_W), jnp.float32),
        pltpu.SemaphoreType.DMA((NB,)),
        pltpu.SemaphoreType.DMA((NB,)),
    ],
)
def _ea_deg_kernel(ea_hbm, dst_hbm, zeros_hbm, out_hbm, acc, didx, rows,
                   isem, ssem):
    cid = lax.axis_index("c")
    sid = lax.axis_index("s")
    wid = sid * NC + cid
    _striped_copy(lambda o, s: zeros_hbm.at[pl.ds(o, s)],
                  lambda o, s: acc.at[pl.ds(o, s)], sid)
    plsc.subcore_barrier()

    def body(g, _):
        idescs = []
        for b in range(NB):
            base = pl.multiple_of(((NB * g + b) * NW + wid) * K, K)
            d1 = pltpu.async_copy(dst_hbm.at[pl.ds(base, K)], didx.at[b],
                                  isem.at[b])
            d2 = pltpu.async_copy(ea_hbm.at[pl.ds(base, K)], rows.at[b],
                                  isem.at[b])
            idescs.append((d1, d2))
        sdescs = []
        for b in range(NB):
            idescs[b][0].wait()
            idescs[b][1].wait()
            sdescs.append(pltpu.async_copy(rows.at[b], acc.at[didx.at[b]],
                                           ssem.at[b], add=True))
        for d in sdescs:
            d.wait()
        return 0

    lax.fori_loop(0, NOUT, body, 0)

    @pl.when(wid < NTAIL)
    def _():
        base = pl.multiple_of((TPW * NW + wid) * K, K)
        pltpu.async_copy(dst_hbm.at[pl.ds(base, K)], didx.at[0],
                         isem.at[0]).wait()
        pltpu.async_copy(ea_hbm.at[pl.ds(base, K)], rows.at[0],
                         isem.at[0]).wait()
        pltpu.sync_copy(rows.at[0], acc.at[didx.at[0]], add=True)

    plsc.subcore_barrier()
    _striped_copy(lambda o, s: acc.at[pl.ds(o, s)],
                  lambda o, s: out_hbm.at[cid].at[pl.ds(o, s)], sid)


@functools.partial(
    pl.kernel,
    out_type=jax.ShapeDtypeStruct((NC, N, H), jnp.float32),
    mesh=_mesh,
    scratch_types=[
        pltpu.VMEM_SHARED((N, H), jnp.float32),
        pltpu.VMEM((2, NB, K), jnp.int32),
        pltpu.VMEM((NB, K), jnp.int32),
        pltpu.VMEM((NB, K, H), jnp.float32),
        pltpu.SemaphoreType.DMA((2, NB)),
        pltpu.SemaphoreType.DMA((NB,)),
        pltpu.SemaphoreType.DMA((NB,)),
        pltpu.SemaphoreType.DMA((NB,)),
    ],
)
def _agg_kernel(y_hbm, src_hbm, dst_hbm, zeros_hbm, out_hbm,
                acc, sidx, didx, rows, isem, gsem, rsem, ssem):
    cid = lax.axis_index("c")
    sid = lax.axis_index("s")
    wid = sid * NC + cid
    _striped_copy(lambda o, s: zeros_hbm.at[pl.ds(o, s)],
                  lambda o, s: acc.at[pl.ds(o, s)], sid)

    def chunk_base(g, b):
        return pl.multiple_of(((NB * g + b) * NW + wid) * K, K)

    def fire_sidx(g, q):
        for b in range(NB):
            pltpu.async_copy(src_hbm.at[pl.ds(chunk_base(g, b), K)],
                             sidx.at[q, b], isem.at[q, b])

    def drain_sidx(q):
        for b in range(NB):
            pltpu.make_async_copy(src_hbm.at[pl.ds(0, K)], sidx.at[q, b],
                                  isem.at[q, b]).wait()

    fire_sidx(0, 0)
    plsc.subcore_barrier()

    def body(g, _):
        def round_(p, q):
            drain_sidx(p)
            gdescs = []
            ddescs = []
            for b in range(NB):
                gdescs.append(pltpu.async_copy(y_hbm.at[sidx.at[p, b]],
                                               rows.at[b], rsem.at[b]))
                ddescs.append(pltpu.async_copy(
                    dst_hbm.at[pl.ds(chunk_base(g, b), K)], didx.at[b],
                    gsem.at[b]))
            gn = jnp.minimum(g + 1, NOUT - 1)
            fire_sidx(gn, q)
            sdescs = []
            for b in range(NB):
                ddescs[b].wait()
                gdescs[b].wait()
                sdescs.append(pltpu.async_copy(rows.at[b],
                                               acc.at[didx.at[b]],
                                               ssem.at[b], add=True))
            for d in sdescs:
                d.wait()

        @pl.when(g % 2 == 0)
        def _():
            round_(0, 1)

        @pl.when(g % 2 == 1)
        def _():
            round_(1, 0)

        return 0

    lax.fori_loop(0, NOUT, body, 0)
    drain_sidx(NOUT % 2)

    @pl.when(wid < NTAIL)
    def _():
        base = pl.multiple_of((TPW * NW + wid) * K, K)
        pltpu.async_copy(src_hbm.at[pl.ds(base, K)], sidx.at[0, 0],
                         isem.at[0, 0]).wait()
        pltpu.async_copy(dst_hbm.at[pl.ds(base, K)], didx.at[0],
                         gsem.at[0]).wait()
        pltpu.async_copy(y_hbm.at[sidx.at[0, 0]], rows.at[0],
                         rsem.at[0]).wait()
        pltpu.sync_copy(rows.at[0], acc.at[didx.at[0]], add=True)

    plsc.subcore_barrier()
    _striped_copy(lambda o, s: acc.at[pl.ds(o, s)],
                  lambda o, s: out_hbm.at[cid].at[pl.ds(o, s)], sid)


RB = 2000
NBLK = N // RB


def _mm_body(x_ref, w_ref, o_ref):
    o_ref[...] = jnp.dot(x_ref[...], w_ref[...],
                         preferred_element_type=jnp.float32)


_mm = pl.pallas_call(
    _mm_body,
    grid=(NBLK,),
    in_specs=[
        pl.BlockSpec((RB, H), lambda i: (i, 0)),
        pl.BlockSpec((H, H), lambda i: (0, 0)),
    ],
    out_specs=pl.BlockSpec((RB, H), lambda i: (i, 0)),
    out_shape=jax.ShapeDtypeStruct((N, H), jnp.float32),
)


def _combine_core(p_ref, x_ref, ea_ref, wiT_ref, weT_ref, b_ref):
    p = p_ref[0] + p_ref[1]
    ea = ea_ref[0] + ea_ref[1]
    deg = ea[:, 16:17]
    eagg = ea[:, :16]
    z = jnp.dot(x_ref[...], wiT_ref[...], preferred_element_type=jnp.float32)
    c = jnp.dot(eagg, weT_ref[...], preferred_element_type=jnp.float32)
    h = p + deg * (z + b_ref[...]) + c
    return jnp.where(h > 0, h, 0.01 * h)


def _combine_mid_body(p_ref, x_ref, ea_ref, wiT_ref, weT_ref, b_ref, wjT_ref,
                      xo_ref, yo_ref):
    xn = _combine_core(p_ref, x_ref, ea_ref, wiT_ref, weT_ref, b_ref)
    xo_ref[...] = xn
    yo_ref[...] = jnp.dot(xn, wjT_ref[...], preferred_element_type=jnp.float32)


def _combine_last_body(p_ref, x_ref, ea_ref, wiT_ref, weT_ref, b_ref, xo_ref):
    xn = _combine_core(p_ref, x_ref, ea_ref, wiT_ref, weT_ref, b_ref)
    xo_ref[...] = jnp.where(xn > 0, xn, 0.01 * xn)


_combine_in_specs = [
    pl.BlockSpec((NC, RB, H), lambda i: (0, i, 0)),
    pl.BlockSpec((RB, H), lambda i: (i, 0)),
    pl.BlockSpec((NC, RB, EA_W), lambda i: (0, i, 0)),
    pl.BlockSpec((H, H), lambda i: (0, 0)),
    pl.BlockSpec((16, H), lambda i: (0, 0)),
    pl.BlockSpec((1, H), lambda i: (0, 0)),
]

_combine_mid = pl.pallas_call(
    _combine_mid_body,
    grid=(NBLK,),
    in_specs=_combine_in_specs + [pl.BlockSpec((H, H), lambda i: (0, 0))],
    out_specs=[
        pl.BlockSpec((RB, H), lambda i: (i, 0)),
        pl.BlockSpec((RB, H), lambda i: (i, 0)),
    ],
    out_shape=[
        jax.ShapeDtypeStruct((N, H), jnp.float32),
        jax.ShapeDtypeStruct((N, H), jnp.float32),
    ],
)

_combine_last = pl.pallas_call(
    _combine_last_body,
    grid=(NBLK,),
    in_specs=_combine_in_specs,
    out_specs=pl.BlockSpec((RB, H), lambda i: (i, 0)),
    out_shape=jax.ShapeDtypeStruct((N, H), jnp.float32),
)


@jax.jit
def kernel(x, edge_index, edge_attr, Wm, bm):
    src = edge_index[0]
    dst = edge_index[1]
    ea_pad = jnp.concatenate(
        [edge_attr,
         jnp.ones((E, 1), jnp.float32),
         jnp.zeros((E, EA_W - 17), jnp.float32)], axis=1)
    zeros_h = jnp.zeros((N, H), jnp.float32)
    zeros_ea = jnp.zeros((N, EA_W), jnp.float32)

    ea_parts = _ea_deg_kernel(ea_pad, dst, zeros_ea)

    wjT = [Wm[l, :, :H].T for l in range(3)]
    wiT = [Wm[l, :, H:2 * H].T for l in range(3)]
    weT = [Wm[l, :, 2 * H:].T for l in range(3)]
    b = [bm[l].reshape(1, H) for l in range(3)]

    y = _mm(x, wjT[0])
    for l in range(3):
        parts = _agg_kernel(y, src, dst, zeros_h)
        if l < 2:
            x, y = _combine_mid(parts, x, ea_parts, wiT[l], weT[l], b[l],
                                wjT[l + 1])
        else:
            x = _combine_last(parts, x, ea_parts, wiT[l], weT[l], b[l])
    return x


# speedup vs baseline: 11.6382x; 1.0008x over previous
import functools
import jax
import jax.numpy as jnp
from jax import lax
from jax.experimental import pallas as pl
from jax.experimental.pallas import tpu as pltpu
from jax.experimental.pallas import tpu_sc as plsc

NC = 2
NS = 16
NW = NC * NS

N = 10000
E = 320000
H = 128
EA_W = 128
K = 128
NCHUNK = E // K
TPW = NCHUNK // NW
NTAIL = NCHUNK - TPW * NW
RPT = 640
RPT_LAST = N - (NS - 1) * RPT


def _striped_copy(src_at, dst_at, sid):
    @pl.when(sid < NS - 1)
    def _():
        off = pl.multiple_of(sid * RPT, 8)
        pltpu.sync_copy(src_at(off, RPT), dst_at(off, RPT))

    @pl.when(sid == NS - 1)
    def _():
        pltpu.sync_copy(src_at((NS - 1) * RPT, RPT_LAST),
                        dst_at((NS - 1) * RPT, RPT_LAST))

_mesh = plsc.VectorSubcoreMesh(core_axis_name="c", subcore_axis_name="s")


NB = 3
NOUT = TPW // NB


@functools.partial(
    pl.kernel,
    out_type=jax.ShapeDtypeStruct((NC, N, EA_W), jnp.float32),
    mesh=_mesh,
    scratch_types=[
        pltpu.VMEM_SHARED((N, EA_W), jnp.float32),
        pltpu.VMEM((2, NB, K), jnp.int32),
        pltpu.VMEM((NB, K, EA_W), jnp.float32),
        pltpu.SemaphoreType.DMA((2, NB)),
        pltpu.SemaphoreType.DMA((NB,)),
        pltpu.SemaphoreType.DMA((NB,)),
    ],
)
def _ea_deg_kernel(ea_hbm, dst_hbm, zeros_hbm, out_hbm, acc, didx, rows,
                   isem, rsem, ssem):
    cid = lax.axis_index("c")
    sid = lax.axis_index("s")
    wid = sid * NC + cid
    _striped_copy(lambda o, s: zeros_hbm.at[pl.ds(o, s)],
                  lambda o, s: acc.at[pl.ds(o, s)], sid)

    def chunk_base(g, b):
        return pl.multiple_of(((NB * g + b) * NW + wid) * K, K)

    def fire_didx(g, q):
        for b in range(NB):
            pltpu.async_copy(dst_hbm.at[pl.ds(chunk_base(g, b), K)],
                             didx.at[q, b], isem.at[q, b])

    def drain_didx(q):
        for b in range(NB):
            pltpu.make_async_copy(dst_hbm.at[pl.ds(0, K)], didx.at[q, b],
                                  isem.at[q, b]).wait()

    fire_didx(0, 0)
    plsc.subcore_barrier()

    def body(g, _):
        def round_(p, q):
            rdescs = []
            for b in range(NB):
                rdescs.append(pltpu.async_copy(
                    ea_hbm.at[pl.ds(chunk_base(g, b), K)], rows.at[b],
                    rsem.at[b]))
            gn = jnp.minimum(g + 1, NOUT - 1)
            fire_didx(gn, q)
            drain_didx(p)
            sdescs = []
            for b in range(NB):
                rdescs[b].wait()
                sdescs.append(pltpu.async_copy(rows.at[b],
                                               acc.at[didx.at[p, b]],
                                               ssem.at[b], add=True))
            for d in sdescs:
                d.wait()

        @pl.when(g % 2 == 0)
        def _():
            round_(0, 1)

        @pl.when(g % 2 == 1)
        def _():
            round_(1, 0)

        return 0

    lax.fori_loop(0, NOUT, body, 0)
    drain_didx(NOUT % 2)

    @pl.when(wid < NTAIL)
    def _():
        base = pl.multiple_of((TPW * NW + wid) * K, K)
        pltpu.async_copy(dst_hbm.at[pl.ds(base, K)], didx.at[0, 0],
                         isem.at[0, 0]).wait()
        pltpu.async_copy(ea_hbm.at[pl.ds(base, K)], rows.at[0],
                         rsem.at[0]).wait()
        pltpu.sync_copy(rows.at[0], acc.at[didx.at[0, 0]], add=True)

    plsc.subcore_barrier()
    _striped_copy(lambda o, s: acc.at[pl.ds(o, s)],
                  lambda o, s: out_hbm.at[cid].at[pl.ds(o, s)], sid)


@functools.partial(
    pl.kernel,
    out_type=jax.ShapeDtypeStruct((NC, N, H), jnp.float32),
    mesh=_mesh,
    scratch_types=[
        pltpu.VMEM_SHARED((N, H), jnp.float32),
        pltpu.VMEM((2, NB, K), jnp.int32),
        pltpu.VMEM((NB, K), jnp.int32),
        pltpu.VMEM((NB, K, H), jnp.float32),
        pltpu.SemaphoreType.DMA((2, NB)),
        pltpu.SemaphoreType.DMA((NB,)),
        pltpu.SemaphoreType.DMA((NB,)),
        pltpu.SemaphoreType.DMA((NB,)),
    ],
)
def _agg_kernel(y_hbm, src_hbm, dst_hbm, zeros_hbm, out_hbm,
                acc, sidx, didx, rows, isem, gsem, rsem, ssem):
    cid = lax.axis_index("c")
    sid = lax.axis_index("s")
    wid = sid * NC + cid
    _striped_copy(lambda o, s: zeros_hbm.at[pl.ds(o, s)],
                  lambda o, s: acc.at[pl.ds(o, s)], sid)

    def chunk_base(g, b):
        return pl.multiple_of(((NB * g + b) * NW + wid) * K, K)

    def fire_sidx(g, q):
        for b in range(NB):
            pltpu.async_copy(src_hbm.at[pl.ds(chunk_base(g, b), K)],
                             sidx.at[q, b], isem.at[q, b])

    def drain_sidx(q):
        for b in range(NB):
            pltpu.make_async_copy(src_hbm.at[pl.ds(0, K)], sidx.at[q, b],
                                  isem.at[q, b]).wait()

    fire_sidx(0, 0)
    plsc.subcore_barrier()

    def body(g, _):
        def round_(p, q):
            drain_sidx(p)
            gdescs = []
            ddescs = []
            for b in range(NB):
                gdescs.append(pltpu.async_copy(y_hbm.at[sidx.at[p, b]],
                                               rows.at[b], rsem.at[b]))
                ddescs.append(pltpu.async_copy(
                    dst_hbm.at[pl.ds(chunk_base(g, b), K)], didx.at[b],
                    gsem.at[b]))
            gn = jnp.minimum(g + 1, NOUT - 1)
            fire_sidx(gn, q)
            sdescs = []
            for b in range(NB):
                ddescs[b].wait()
                gdescs[b].wait()
                sdescs.append(pltpu.async_copy(rows.at[b],
                                               acc.at[didx.at[b]],
                                               ssem.at[b], add=True))
            for d in sdescs:
                d.wait()

        @pl.when(g % 2 == 0)
        def _():
            round_(0, 1)

        @pl.when(g % 2 == 1)
        def _():
            round_(1, 0)

        return 0

    lax.fori_loop(0, NOUT, body, 0)
    drain_sidx(NOUT % 2)

    @pl.when(wid < NTAIL)
    def _():
        base = pl.multiple_of((TPW * NW + wid) * K, K)
        pltpu.async_copy(src_hbm.at[pl.ds(base, K)], sidx.at[0, 0],
                         isem.at[0, 0]).wait()
        pltpu.async_copy(dst_hbm.at[pl.ds(base, K)], didx.at[0],
                         gsem.at[0]).wait()
        pltpu.async_copy(y_hbm.at[sidx.at[0, 0]], rows.at[0],
                         rsem.at[0]).wait()
        pltpu.sync_copy(rows.at[0], acc.at[didx.at[0]], add=True)

    plsc.subcore_barrier()
    _striped_copy(lambda o, s: acc.at[pl.ds(o, s)],
                  lambda o, s: out_hbm.at[cid].at[pl.ds(o, s)], sid)


RB = 2000
NBLK = N // RB


def _mm_body(x_ref, w_ref, o_ref):
    o_ref[...] = jnp.dot(x_ref[...], w_ref[...],
                         preferred_element_type=jnp.float32)


_mm = pl.pallas_call(
    _mm_body,
    grid=(NBLK,),
    in_specs=[
        pl.BlockSpec((RB, H), lambda i: (i, 0)),
        pl.BlockSpec((H, H), lambda i: (0, 0)),
    ],
    out_specs=pl.BlockSpec((RB, H), lambda i: (i, 0)),
    out_shape=jax.ShapeDtypeStruct((N, H), jnp.float32),
)


def _combine_core(p_ref, x_ref, ea_ref, wiT_ref, weT_ref, b_ref):
    p = p_ref[0] + p_ref[1]
    ea = ea_ref[0] + ea_ref[1]
    deg = ea[:, 16:17]
    eagg = ea[:, :16]
    z = jnp.dot(x_ref[...], wiT_ref[...], preferred_element_type=jnp.float32)
    c = jnp.dot(eagg, weT_ref[...], preferred_element_type=jnp.float32)
    h = p + deg * (z + b_ref[...]) + c
    return jnp.where(h > 0, h, 0.01 * h)


def _combine_mid_body(p_ref, x_ref, ea_ref, wiT_ref, weT_ref, b_ref, wjT_ref,
                      xo_ref, yo_ref):
    xn = _combine_core(p_ref, x_ref, ea_ref, wiT_ref, weT_ref, b_ref)
    xo_ref[...] = xn
    yo_ref[...] = jnp.dot(xn, wjT_ref[...], preferred_element_type=jnp.float32)


def _combine_last_body(p_ref, x_ref, ea_ref, wiT_ref, weT_ref, b_ref, xo_ref):
    xn = _combine_core(p_ref, x_ref, ea_ref, wiT_ref, weT_ref, b_ref)
    xo_ref[...] = jnp.where(xn > 0, xn, 0.01 * xn)


_combine_in_specs = [
    pl.BlockSpec((NC, RB, H), lambda i: (0, i, 0)),
    pl.BlockSpec((RB, H), lambda i: (i, 0)),
    pl.BlockSpec((NC, RB, EA_W), lambda i: (0, i, 0)),
    pl.BlockSpec((H, H), lambda i: (0, 0)),
    pl.BlockSpec((16, H), lambda i: (0, 0)),
    pl.BlockSpec((1, H), lambda i: (0, 0)),
]

_combine_mid = pl.pallas_call(
    _combine_mid_body,
    grid=(NBLK,),
    in_specs=_combine_in_specs + [pl.BlockSpec((H, H), lambda i: (0, 0))],
    out_specs=[
        pl.BlockSpec((RB, H), lambda i: (i, 0)),
        pl.BlockSpec((RB, H), lambda i: (i, 0)),
    ],
    out_shape=[
        jax.ShapeDtypeStruct((N, H), jnp.float32),
        jax.ShapeDtypeStruct((N, H), jnp.float32),
    ],
)

_combine_last = pl.pallas_call(
    _combine_last_body,
    grid=(NBLK,),
    in_specs=_combine_in_specs,
    out_specs=pl.BlockSpec((RB, H), lambda i: (i, 0)),
    out_shape=jax.ShapeDtypeStruct((N, H), jnp.float32),
)


@jax.jit
def kernel(x, edge_index, edge_attr, Wm, bm):
    src = edge_index[0]
    dst = edge_index[1]
    ea_pad = jnp.concatenate(
        [edge_attr,
         jnp.ones((E, 1), jnp.float32),
         jnp.zeros((E, EA_W - 17), jnp.float32)], axis=1)
    zeros_h = jnp.zeros((N, H), jnp.float32)
    zeros_ea = jnp.zeros((N, EA_W), jnp.float32)

    ea_parts = _ea_deg_kernel(ea_pad, dst, zeros_ea)

    wjT = [Wm[l, :, :H].T for l in range(3)]
    wiT = [Wm[l, :, H:2 * H].T for l in range(3)]
    weT = [Wm[l, :, 2 * H:].T for l in range(3)]
    b = [bm[l].reshape(1, H) for l in range(3)]

    y = _mm(x, wjT[0])
    for l in range(3):
        parts = _agg_kernel(y, src, dst, zeros_h)
        if l < 2:
            x, y = _combine_mid(parts, x, ea_parts, wiT[l], weT[l], b[l],
                                wjT[l + 1])
        else:
            x = _combine_last(parts, x, ea_parts, wiT[l], weT[l], b[l])
    return x
